# baseline (speedup 1.0000x reference)
import functools

import jax
import jax.numpy as jnp
from jax import lax
from jax.experimental import pallas as pl
from jax.experimental.pallas import tpu as pltpu
from jax.experimental.pallas import tpu_sc as plsc

_V = 1000
_VP = 1024
_NB = 4096
_L = 20
_NC = 2
_NS = 16
_TPW = 4
_NT = _V // 8
_LANES = 16
_G = _NB // _LANES


def _mm_body(w_ref, emb_ref, b_ref, out_ref):
    out_ref[...] = lax.dot_general(
        w_ref[...], emb_ref[...], (((1,), (1,)), ((), ())),
        preferred_element_type=jnp.float32) + b_ref[...]


def _make_table_t(lin_w, emb_pad, lin_b_col):
    return pl.pallas_call(
        _mm_body,
        out_shape=jax.ShapeDtypeStruct((_V, _VP), jnp.float32),
    )(lin_w, emb_pad, lin_b_col)


@functools.lru_cache(maxsize=1)
def _make_lookup():
    mesh = plsc.VectorSubcoreMesh(core_axis_name="c", subcore_axis_name="s")

    @functools.partial(
        pl.kernel,
        mesh=mesh,
        out_type=jax.ShapeDtypeStruct((_L, _V, _NB), jnp.float32),
        scratch_types=[
            pltpu.VMEM((8 * _TPW, _VP), jnp.float32),
            pltpu.VMEM((_NB,), jnp.int32),
            pltpu.VMEM((_NB,), jnp.int32),
            pltpu.VMEM((1, 8, _NB), jnp.float32),
            pltpu.VMEM((1, 8, _NB), jnp.float32),
            pltpu.SemaphoreType.DMA,
            pltpu.SemaphoreType.DMA,
            pltpu.SemaphoreType.DMA,
        ],
        compiler_params=pltpu.CompilerParams(
            use_tc_tiling_on_sc=True, needs_layout_passes=False),
    )
    def _lookup(mt_hbm, idx_hbm, out_hbm, slabs, ib0, ib1, ob0, ob1,
                sem0, sem1, isem):
        wid = lax.axis_index("s") * _NC + lax.axis_index("c")
        obufs = (ob0, ob1)
        sems = (sem0, sem1)

        def _wait_store(p):
            pltpu.make_async_copy(
                obufs[p],
                out_hbm.at[pl.ds(0, 1), pl.ds(0, 8), :],
                sems[p],
            ).wait()

        ts = []
        for k in range(_TPW):
            t = jnp.minimum(_TPW * wid + k, _NT - 1)
            ts.append(t)
            pltpu.sync_copy(mt_hbm.at[pl.ds(8 * t, 8), :],
                            slabs.at[pl.ds(8 * k, 8), :])

        def _units(l, ids_v):
            for k in range(_TPW):
                p = k % 2
                if k < 2:
                    @pl.when(l > 0)
                    def _():
                        _wait_store(p)
                else:
                    _wait_store(p)
                ob = obufs[p]

                @plsc.parallel_loop(0, _G, 1, unroll=8)
                def per_g(g):
                    idv = ids_v[pl.ds(g * _LANES, _LANES)]
                    for v in range(8):
                        vec = plsc.load_gather(
                            slabs,
                            [jnp.full((_LANES,), 8 * k + v, jnp.int32), idv])
                        ob[0, v, pl.ds(g * _LANES, _LANES)] = vec

                pltpu.async_copy(
                    ob,
                    out_hbm.at[pl.ds(l, 1), pl.ds(8 * ts[k], 8), :],
                    sems[p],
                )

        pltpu.sync_copy(idx_hbm.at[pl.ds(0, _NB)], ib0)

        def per_l2(l2, carry):
            l = l2 * 2
            a1 = pltpu.async_copy(
                idx_hbm.at[pl.ds((l + 1) * _NB, _NB)], ib1, isem)
            _units(l, ib0)
            a1.wait()
            nxt = jnp.minimum(l + 2, _L - 1) * _NB
            a0 = pltpu.async_copy(idx_hbm.at[pl.ds(nxt, _NB)], ib0, isem)
            _units(l + 1, ib1)
            a0.wait()
            return carry

        lax.fori_loop(0, _L // 2, per_l2, 0)
        _wait_store(0)
        _wait_store(1)

    return _lookup


def kernel(input_ids, emb_table, lin_w, lin_b):
    emb_pad = jnp.pad(emb_table, ((0, _VP - _V), (0, 0)))
    b_col = lin_b.reshape(_V, 1)
    mt = _make_table_t(lin_w, emb_pad, b_col)
    ids_t = input_ids.T.reshape(-1).astype(jnp.int32)
    out = _make_lookup()(mt, ids_t)
    return out.transpose(2, 0, 1)
